# Initial kernel scaffold; baseline (speedup 1.0000x reference)
import jax
import jax.numpy as jnp
from jax.experimental import pallas as pl


def kernel(x, edge_index, batch, len_longest_path, io_ratio, W_enc, b_enc, gin_W1, gin_b1, gin_W2, gin_b2, vn_W1, vn_b1, vn_W2, vn_b2, W_pred, b_pred, W_level, b_level, W_io, b_io, W_comb, b_comb):
    raise NotImplementedError("write your pallas kernel here")


# speedup vs baseline: 3.3261x; 3.3261x over previous
import functools
import math

import jax
import jax.numpy as jnp
from jax import lax
from jax.experimental import pallas as pl
from jax.experimental.pallas import tpu as pltpu
from jax.experimental.pallas import tpu_sc as plsc

EMB = 64
NL = 5
NG = 512
N_NODES = 50000
N_EDGES = 800000
D_FEAT = 128

BLK = 1024
NP = 50176
NBLK = NP // BLK
HHALF = NP // 2
TRASH = 128
ACC_ROWS = HHALF + TRASH
ZSTRIPE = ACC_ROWS // 16
OSTRIPE = HHALF // 16
EROWS = N_EDGES // 128
EITERS = (EROWS + 15) // 16

BN = 1.0 / math.sqrt(1.0 + 1e-5)

f32 = jnp.float32


def _sc_agg_body(r_hbm, src_hbm, dst_hbm, zeros_hbm, out_hbm,
                 sidx, draw, didx, rows, acc, sem):
    cid = lax.axis_index("c")
    sid = lax.axis_index("s")
    pltpu.sync_copy(zeros_hbm, acc.at[pl.ds(sid * ZSTRIPE, ZSTRIPE)])
    plsc.subcore_barrier()
    base = cid * HHALF

    @pl.loop(0, EITERS)
    def _edge_iter(j):
        row = sid + j * 16

        @pl.when(row < EROWS)
        def _():
            pltpu.sync_copy(src_hbm.at[row], sidx)
            pltpu.sync_copy(dst_hbm.at[row], draw)
            for k in range(8):
                d = draw[pl.ds(k * 16, 16)]
                loc = d - base
                ok = (loc >= 0) & (loc < HHALF)
                alt = HHALF + (d & (TRASH - 1))
                didx[pl.ds(k * 16, 16)] = jnp.where(ok, loc, alt)
            pltpu.async_copy(r_hbm.at[sidx], rows, sem).wait()
            pltpu.sync_copy(rows, acc.at[didx], add=True)

    plsc.subcore_barrier()
    pltpu.sync_copy(acc.at[pl.ds(sid * OSTRIPE, OSTRIPE)],
                    out_hbm.at[pl.ds(cid * HHALF + sid * OSTRIPE, OSTRIPE)])


@functools.cache
def _sc_agg_kernel():
    return pl.kernel(
        _sc_agg_body,
        out_type=jax.ShapeDtypeStruct((NP, EMB), f32),
        mesh=plsc.VectorSubcoreMesh(core_axis_name="c", subcore_axis_name="s"),
        scratch_types=[
            pltpu.VMEM((128,), jnp.int32),
            pltpu.VMEM((128,), jnp.int32),
            pltpu.VMEM((128,), jnp.int32),
            pltpu.VMEM((128, EMB), f32),
            pltpu.VMEM_SHARED((ACC_ROWS, EMB), f32),
            pltpu.SemaphoreType.DMA,
        ],
        compiler_params=pltpu.CompilerParams(use_tc_tiling_on_sc=False),
    )


def _sc_agg(r, src2d, dst2d, zeros):
    return _sc_agg_kernel()(r, src2d, dst2d, zeros)


def _onehot(batch_blk):
    gi = lax.broadcasted_iota(jnp.int32, (BLK, NG), 1)
    return (batch_blk[:, None] == gi).astype(f32)


def _enc_body(x_ref, b3_ref, W_ref, b_ref, hin_ref, r_ref, vnsum_ref):
    i = pl.program_id(0)
    hin = jnp.dot(x_ref[...], W_ref[...], preferred_element_type=f32) + b_ref[...]
    hin_ref[...] = hin
    r_ref[...] = jnp.maximum(hin, 0.0)
    oh = _onehot(b3_ref[0, 0])

    @pl.when(i == 0)
    def _():
        vnsum_ref[...] = jnp.zeros_like(vnsum_ref)

    vnsum_ref[...] += lax.dot_general(oh, hin, (((0,), (0,)), ((), ())),
                                      preferred_element_type=f32)


def _a_body(h_ref, b3_ref, vn_ref, hin_ref, r_ref, vnsum_ref):
    i = pl.program_id(0)
    oh = _onehot(b3_ref[0, 0])
    hin = h_ref[...] + jnp.dot(oh, vn_ref[...], preferred_element_type=f32)
    hin_ref[...] = hin
    r_ref[...] = jnp.maximum(hin, 0.0)

    @pl.when(i == 0)
    def _():
        vnsum_ref[...] = jnp.zeros_like(vnsum_ref)

    vnsum_ref[...] += lax.dot_general(oh, hin, (((0,), (0,)), ((), ())),
                                      preferred_element_type=f32)


def _mlp(z, W1, b1, W2, b2):
    z1 = jnp.dot(z, W1, preferred_element_type=f32) + b1
    z2 = jnp.maximum(z1 * BN, 0.0)
    z3 = jnp.dot(z2, W2, preferred_element_type=f32) + b2
    return z3 * BN


def _b_body(hin_ref, agg_ref, W1_ref, b1_ref, W2_ref, b2_ref, h_ref):
    h = _mlp(hin_ref[...] + agg_ref[...],
             W1_ref[...], b1_ref[...], W2_ref[...], b2_ref[...])
    h_ref[...] = jnp.maximum(h, 0.0)


def _b_last_body(hin_ref, agg_ref, b3_ref, W1_ref, b1_ref, W2_ref, b2_ref,
                 pool_ref):
    i = pl.program_id(0)
    h = _mlp(hin_ref[...] + agg_ref[...],
             W1_ref[...], b1_ref[...], W2_ref[...], b2_ref[...])
    oh = _onehot(b3_ref[0, 0])

    @pl.when(i == 0)
    def _():
        pool_ref[...] = jnp.zeros_like(pool_ref)

    pool_ref[...] += lax.dot_general(oh, h, (((0,), (0,)), ((), ())),
                                     preferred_element_type=f32)


def _c_body(vnsum_ref, vn_ref, W1_ref, b1_ref, W2_ref, b2_ref, out_ref):
    v = _mlp(vnsum_ref[...] + vn_ref[...],
             W1_ref[...], b1_ref[...], W2_ref[...], b2_ref[...])
    out_ref[...] = jnp.maximum(v, 0.0)


def _head_body(pool_ref, Wp_ref, bp_ref, llp_ref, Wl_ref, bl_ref,
               io_ref, Wio_ref, bio_ref, Wc1_ref, Wc2_ref, Wc3_ref, bc_ref,
               out_ref):
    hg = jnp.dot(pool_ref[...], Wp_ref[...], preferred_element_type=f32) \
        + bp_ref[...]
    lvl = jnp.maximum((llp_ref[...] * Wl_ref[...] + bl_ref[...]) * BN, 0.0)
    io = jnp.maximum((io_ref[...] * Wio_ref[...] + bio_ref[...]) * BN, 0.0)
    comb = (jnp.dot(hg, Wc1_ref[...], preferred_element_type=f32)
            + jnp.dot(io, Wc2_ref[...], preferred_element_type=f32)
            + jnp.dot(lvl, Wc3_ref[...], preferred_element_type=f32)
            + bc_ref[...])
    out_ref[...] = jnp.maximum(comb * BN, 0.0)


def _nspec(cols):
    return pl.BlockSpec((BLK, cols), lambda i: (i, 0))


def _wspec(r, c):
    return pl.BlockSpec((r, c), lambda i: (0, 0))


_B3SPEC = pl.BlockSpec((1, 1, BLK), lambda i: (i, 0, 0))
_GSPEC = pl.BlockSpec((NG, EMB), lambda i: (0, 0))

_node2 = [jax.ShapeDtypeStruct((NP, EMB), f32)] * 2
_gout = jax.ShapeDtypeStruct((NG, EMB), f32)

_enc_call = pl.pallas_call(
    _enc_body, grid=(NBLK,),
    in_specs=[_nspec(D_FEAT), _B3SPEC, _wspec(D_FEAT, EMB), _wspec(1, EMB)],
    out_specs=[_nspec(EMB), _nspec(EMB), _GSPEC],
    out_shape=_node2 + [_gout],
)

_a_call = pl.pallas_call(
    _a_body, grid=(NBLK,),
    in_specs=[_nspec(EMB), _B3SPEC, _GSPEC],
    out_specs=[_nspec(EMB), _nspec(EMB), _GSPEC],
    out_shape=_node2 + [_gout],
)

_b_call = pl.pallas_call(
    _b_body, grid=(NBLK,),
    in_specs=[_nspec(EMB), _nspec(EMB), _wspec(EMB, 2 * EMB),
              _wspec(1, 2 * EMB), _wspec(2 * EMB, EMB), _wspec(1, EMB)],
    out_specs=[_nspec(EMB)],
    out_shape=[jax.ShapeDtypeStruct((NP, EMB), f32)],
)

_b_last_call = pl.pallas_call(
    _b_last_body, grid=(NBLK,),
    in_specs=[_nspec(EMB), _nspec(EMB), _B3SPEC, _wspec(EMB, 2 * EMB),
              _wspec(1, 2 * EMB), _wspec(2 * EMB, EMB), _wspec(1, EMB)],
    out_specs=[_GSPEC],
    out_shape=[_gout],
)

_c_call = pl.pallas_call(
    _c_body,
    out_shape=[_gout],
)

_head_call = pl.pallas_call(
    _head_body,
    out_shape=jax.ShapeDtypeStruct((NG, EMB), f32),
)


def kernel(x, edge_index, batch, len_longest_path, io_ratio, W_enc, b_enc,
           gin_W1, gin_b1, gin_W2, gin_b2, vn_W1, vn_b1, vn_W2, vn_b2,
           W_pred, b_pred, W_level, b_level, W_io, b_io, W_comb, b_comb):
    x_p = jnp.pad(x, ((0, NP - N_NODES), (0, 0)))
    batch_p = jnp.pad(batch.astype(jnp.int32), (0, NP - N_NODES),
                      constant_values=NG).reshape(NBLK, 1, BLK)
    src2d = edge_index[0].astype(jnp.int32).reshape(EROWS, 128)
    dst2d = edge_index[1].astype(jnp.int32).reshape(EROWS, 128)
    zeros = jnp.zeros((ZSTRIPE, EMB), f32)

    h_in, r, vnsum = _enc_call(x_p, batch_p, W_enc, b_enc.reshape(1, EMB))
    vn = jnp.zeros((NG, EMB), f32)
    for l in range(NL):
        agg = _sc_agg(r, src2d, dst2d, zeros)
        if l < NL - 1:
            (vn,) = _c_call(vnsum, vn, vn_W1[l], vn_b1[l].reshape(1, 2 * EMB),
                            vn_W2[l], vn_b2[l].reshape(1, EMB))
            (h,) = _b_call(h_in, agg, gin_W1[l], gin_b1[l].reshape(1, 2 * EMB),
                           gin_W2[l], gin_b2[l].reshape(1, EMB))
            h_in, r, vnsum = _a_call(h, batch_p, vn)
        else:
            (pool,) = _b_last_call(h_in, agg, batch_p, gin_W1[l],
                                   gin_b1[l].reshape(1, 2 * EMB), gin_W2[l],
                                   gin_b2[l].reshape(1, EMB))
    return _head_call(pool, W_pred, b_pred.reshape(1, EMB),
                      len_longest_path.reshape(NG, 1), W_level,
                      b_level.reshape(1, EMB), io_ratio.reshape(NG, 1),
                      W_io, b_io.reshape(1, EMB), W_comb[:EMB],
                      W_comb[EMB:2 * EMB], W_comb[2 * EMB:],
                      b_comb.reshape(1, EMB))


# speedup vs baseline: 5.4713x; 1.6449x over previous
import functools
import math

import jax
import jax.numpy as jnp
from jax import lax
from jax.experimental import pallas as pl
from jax.experimental.pallas import tpu as pltpu
from jax.experimental.pallas import tpu_sc as plsc

EMB = 64
NL = 5
NG = 512
N_NODES = 50000
N_EDGES = 800000
D_FEAT = 128

BLK = 1024
NP = 50176
NBLK = NP // BLK
HHALF = NP // 2
TRASH = 128
ACC_ROWS = HHALF + TRASH
ZSTRIPE = ACC_ROWS // 16
OSTRIPE = HHALF // 16
EPT = 50176
NE_PAD = 16 * EPT
EPAD = NE_PAD - N_EDGES
CHUNK = 112
NCH = EPT // CHUNK

BN = 1.0 / math.sqrt(1.0 + 1e-5)

f32 = jnp.float32


SLOTS = 4


def _sc_dloc_body(dst_hbm, dloc0_hbm, dloc1_hbm, buf):
    cid = lax.axis_index("c")
    sid = lax.axis_index("s")
    base = cid * HHALF
    span = EPT
    pltpu.sync_copy(dst_hbm.at[pl.ds(sid * span, span)], buf)

    @pl.loop(0, span // 16)
    def _(i):
        v = buf[pl.ds(i * 16, 16)]
        loc = v - base
        ok = (loc >= 0) & (loc < HHALF)
        buf[pl.ds(i * 16, 16)] = jnp.where(ok, loc, HHALF + (v & (TRASH - 1)))

    @pl.when(cid == 0)
    def _():
        pltpu.sync_copy(buf, dloc0_hbm.at[pl.ds(sid * span, span)])

    @pl.when(cid == 1)
    def _():
        pltpu.sync_copy(buf, dloc1_hbm.at[pl.ds(sid * span, span)])


@functools.cache
def _sc_dloc_kernel():
    return pl.kernel(
        _sc_dloc_body,
        out_type=[jax.ShapeDtypeStruct((NE_PAD,), jnp.int32)] * 2,
        mesh=plsc.VectorSubcoreMesh(core_axis_name="c", subcore_axis_name="s"),
        scratch_types=[pltpu.VMEM((EPT,), jnp.int32)],
        compiler_params=pltpu.CompilerParams(use_tc_tiling_on_sc=False),
    )


def _sc_agg_body(r_hbm, src_hbm, dloc0_hbm, dloc1_hbm, zeros_hbm, out_hbm,
                 sidx, didx, rows, acc, lsem, dsem, gsem, ssem):
    cid = lax.axis_index("c")
    sid = lax.axis_index("s")
    pltpu.sync_copy(zeros_hbm, acc.at[pl.ds(sid * ZSTRIPE, ZSTRIPE)])
    start = sid * EPT
    plsc.subcore_barrier()

    @pl.loop(0, NCH // SLOTS)
    def _chunk_iter(j):
        ld, dd, gd, sd = [], [], [], []
        bsl = [pl.ds(b * CHUNK, CHUNK) for b in range(SLOTS)]
        for b in range(SLOTS):
            esl = pl.ds(start + (j * SLOTS + b) * CHUNK, CHUNK)
            ld.append(pltpu.async_copy(src_hbm.at[esl], sidx.at[bsl[b]],
                                       lsem.at[b]))

            @pl.when(cid == 0)
            def _(esl=esl, b=b):
                pltpu.async_copy(dloc0_hbm.at[esl], didx.at[bsl[b]],
                                 dsem.at[b])

            @pl.when(cid == 1)
            def _(esl=esl, b=b):
                pltpu.async_copy(dloc1_hbm.at[esl], didx.at[bsl[b]],
                                 dsem.at[b])

            dd.append(pltpu.make_async_copy(dloc0_hbm.at[esl],
                                            didx.at[bsl[b]], dsem.at[b]))
        for b in range(SLOTS):
            ld[b].wait()
            gd.append(pltpu.async_copy(r_hbm.at[sidx.at[bsl[b]]],
                                       rows.at[bsl[b]], gsem.at[b]))
        for b in range(SLOTS):
            gd[b].wait()
            dd[b].wait()
            sd.append(pltpu.async_copy(rows.at[bsl[b]],
                                       acc.at[didx.at[bsl[b]]],
                                       ssem.at[b], add=True))
        for b in range(SLOTS):
            sd[b].wait()

    plsc.subcore_barrier()
    pltpu.sync_copy(acc.at[pl.ds(sid * OSTRIPE, OSTRIPE)],
                    out_hbm.at[pl.ds(cid * HHALF + sid * OSTRIPE, OSTRIPE)])


@functools.cache
def _sc_agg_kernel():
    return pl.kernel(
        _sc_agg_body,
        out_type=jax.ShapeDtypeStruct((NP, EMB), f32),
        mesh=plsc.VectorSubcoreMesh(core_axis_name="c", subcore_axis_name="s"),
        scratch_types=[
            pltpu.VMEM((SLOTS * CHUNK,), jnp.int32),
            pltpu.VMEM((SLOTS * CHUNK,), jnp.int32),
            pltpu.VMEM((SLOTS * CHUNK, EMB), f32),
            pltpu.VMEM_SHARED((ACC_ROWS, EMB), f32),
            pltpu.SemaphoreType.DMA((SLOTS,)),
            pltpu.SemaphoreType.DMA((SLOTS,)),
            pltpu.SemaphoreType.DMA((SLOTS,)),
            pltpu.SemaphoreType.DMA((SLOTS,)),
        ],
        compiler_params=pltpu.CompilerParams(use_tc_tiling_on_sc=False),
    )


def _sc_agg(r, src2d, dloc0, dloc1, zeros):
    return _sc_agg_kernel()(r, src2d, dloc0, dloc1, zeros)


def _onehot(batch_blk):
    gi = lax.broadcasted_iota(jnp.int32, (BLK, NG), 1)
    return (batch_blk[:, None] == gi).astype(f32)


def _enc_body(x_ref, b3_ref, W_ref, b_ref, hin_ref, r_ref, vnsum_ref):
    i = pl.program_id(0)
    hin = jnp.dot(x_ref[...], W_ref[...], preferred_element_type=f32) + b_ref[...]
    hin_ref[...] = hin
    r_ref[...] = jnp.maximum(hin, 0.0)
    oh = _onehot(b3_ref[0, 0])

    @pl.when(i == 0)
    def _():
        vnsum_ref[...] = jnp.zeros_like(vnsum_ref)

    vnsum_ref[...] += lax.dot_general(oh, hin, (((0,), (0,)), ((), ())),
                                      preferred_element_type=f32)


def _a_body(h_ref, b3_ref, vn_ref, hin_ref, r_ref, vnsum_ref):
    i = pl.program_id(0)
    oh = _onehot(b3_ref[0, 0])
    hin = h_ref[...] + jnp.dot(oh, vn_ref[...], preferred_element_type=f32)
    hin_ref[...] = hin
    r_ref[...] = jnp.maximum(hin, 0.0)

    @pl.when(i == 0)
    def _():
        vnsum_ref[...] = jnp.zeros_like(vnsum_ref)

    vnsum_ref[...] += lax.dot_general(oh, hin, (((0,), (0,)), ((), ())),
                                      preferred_element_type=f32)


def _mlp(z, W1, b1, W2, b2):
    z1 = jnp.dot(z, W1, preferred_element_type=f32) + b1
    z2 = jnp.maximum(z1 * BN, 0.0)
    z3 = jnp.dot(z2, W2, preferred_element_type=f32) + b2
    return z3 * BN


def _b_body(hin_ref, agg_ref, W1_ref, b1_ref, W2_ref, b2_ref, h_ref):
    h = _mlp(hin_ref[...] + agg_ref[...],
             W1_ref[...], b1_ref[...], W2_ref[...], b2_ref[...])
    h_ref[...] = jnp.maximum(h, 0.0)


def _b_last_body(hin_ref, agg_ref, b3_ref, W1_ref, b1_ref, W2_ref, b2_ref,
                 pool_ref):
    i = pl.program_id(0)
    h = _mlp(hin_ref[...] + agg_ref[...],
             W1_ref[...], b1_ref[...], W2_ref[...], b2_ref[...])
    oh = _onehot(b3_ref[0, 0])

    @pl.when(i == 0)
    def _():
        pool_ref[...] = jnp.zeros_like(pool_ref)

    pool_ref[...] += lax.dot_general(oh, h, (((0,), (0,)), ((), ())),
                                     preferred_element_type=f32)


def _c_body(vnsum_ref, vn_ref, W1_ref, b1_ref, W2_ref, b2_ref, out_ref):
    v = _mlp(vnsum_ref[...] + vn_ref[...],
             W1_ref[...], b1_ref[...], W2_ref[...], b2_ref[...])
    out_ref[...] = jnp.maximum(v, 0.0)


def _head_body(pool_ref, Wp_ref, bp_ref, llp_ref, Wl_ref, bl_ref,
               io_ref, Wio_ref, bio_ref, Wc1_ref, Wc2_ref, Wc3_ref, bc_ref,
               out_ref):
    hg = jnp.dot(pool_ref[...], Wp_ref[...], preferred_element_type=f32) \
        + bp_ref[...]
    lvl = jnp.maximum((llp_ref[...] * Wl_ref[...] + bl_ref[...]) * BN, 0.0)
    io = jnp.maximum((io_ref[...] * Wio_ref[...] + bio_ref[...]) * BN, 0.0)
    comb = (jnp.dot(hg, Wc1_ref[...], preferred_element_type=f32)
            + jnp.dot(io, Wc2_ref[...], preferred_element_type=f32)
            + jnp.dot(lvl, Wc3_ref[...], preferred_element_type=f32)
            + bc_ref[...])
    out_ref[...] = jnp.maximum(comb * BN, 0.0)


def _nspec(cols):
    return pl.BlockSpec((BLK, cols), lambda i: (i, 0))


def _wspec(r, c):
    return pl.BlockSpec((r, c), lambda i: (0, 0))


_B3SPEC = pl.BlockSpec((1, 1, BLK), lambda i: (i, 0, 0))
_GSPEC = pl.BlockSpec((NG, EMB), lambda i: (0, 0))

_node2 = [jax.ShapeDtypeStruct((NP, EMB), f32)] * 2
_gout = jax.ShapeDtypeStruct((NG, EMB), f32)

_enc_call = pl.pallas_call(
    _enc_body, grid=(NBLK,),
    in_specs=[_nspec(D_FEAT), _B3SPEC, _wspec(D_FEAT, EMB), _wspec(1, EMB)],
    out_specs=[_nspec(EMB), _nspec(EMB), _GSPEC],
    out_shape=_node2 + [_gout],
)

_a_call = pl.pallas_call(
    _a_body, grid=(NBLK,),
    in_specs=[_nspec(EMB), _B3SPEC, _GSPEC],
    out_specs=[_nspec(EMB), _nspec(EMB), _GSPEC],
    out_shape=_node2 + [_gout],
)

_b_call = pl.pallas_call(
    _b_body, grid=(NBLK,),
    in_specs=[_nspec(EMB), _nspec(EMB), _wspec(EMB, 2 * EMB),
              _wspec(1, 2 * EMB), _wspec(2 * EMB, EMB), _wspec(1, EMB)],
    out_specs=[_nspec(EMB)],
    out_shape=[jax.ShapeDtypeStruct((NP, EMB), f32)],
)

_b_last_call = pl.pallas_call(
    _b_last_body, grid=(NBLK,),
    in_specs=[_nspec(EMB), _nspec(EMB), _B3SPEC, _wspec(EMB, 2 * EMB),
              _wspec(1, 2 * EMB), _wspec(2 * EMB, EMB), _wspec(1, EMB)],
    out_specs=[_GSPEC],
    out_shape=[_gout],
)

_c_call = pl.pallas_call(
    _c_body,
    out_shape=[_gout],
)

_head_call = pl.pallas_call(
    _head_body,
    out_shape=jax.ShapeDtypeStruct((NG, EMB), f32),
)


def kernel(x, edge_index, batch, len_longest_path, io_ratio, W_enc, b_enc,
           gin_W1, gin_b1, gin_W2, gin_b2, vn_W1, vn_b1, vn_W2, vn_b2,
           W_pred, b_pred, W_level, b_level, W_io, b_io, W_comb, b_comb):
    x_p = jnp.pad(x, ((0, NP - N_NODES), (0, 0)))
    batch_p = jnp.pad(batch.astype(jnp.int32), (0, NP - N_NODES),
                      constant_values=NG).reshape(NBLK, 1, BLK)
    src_flat = jnp.pad(edge_index[0].astype(jnp.int32), (0, EPAD))
    dst_flat = jnp.pad(edge_index[1].astype(jnp.int32), (0, EPAD),
                       constant_values=NP - 1)
    zeros = jnp.zeros((ZSTRIPE, EMB), f32)
    dloc0, dloc1 = _sc_dloc_kernel()(dst_flat)

    h_in, r, vnsum = _enc_call(x_p, batch_p, W_enc, b_enc.reshape(1, EMB))
    vn = jnp.zeros((NG, EMB), f32)
    for l in range(NL):
        agg = _sc_agg(r, src_flat, dloc0, dloc1, zeros)
        if l < NL - 1:
            (vn,) = _c_call(vnsum, vn, vn_W1[l], vn_b1[l].reshape(1, 2 * EMB),
                            vn_W2[l], vn_b2[l].reshape(1, EMB))
            (h,) = _b_call(h_in, agg, gin_W1[l], gin_b1[l].reshape(1, 2 * EMB),
                           gin_W2[l], gin_b2[l].reshape(1, EMB))
            h_in, r, vnsum = _a_call(h, batch_p, vn)
        else:
            (pool,) = _b_last_call(h_in, agg, batch_p, gin_W1[l],
                                   gin_b1[l].reshape(1, 2 * EMB), gin_W2[l],
                                   gin_b2[l].reshape(1, EMB))
    return _head_call(pool, W_pred, b_pred.reshape(1, EMB),
                      len_longest_path.reshape(NG, 1), W_level,
                      b_level.reshape(1, EMB), io_ratio.reshape(NG, 1),
                      W_io, b_io.reshape(1, EMB), W_comb[:EMB],
                      W_comb[EMB:2 * EMB], W_comb[2 * EMB:],
                      b_comb.reshape(1, EMB))


# speedup vs baseline: 6.0792x; 1.1111x over previous
import functools
import math

import jax
import jax.numpy as jnp
from jax import lax
from jax.experimental import pallas as pl
from jax.experimental.pallas import tpu as pltpu
from jax.experimental.pallas import tpu_sc as plsc

EMB = 64
NL = 5
NG = 512
N_NODES = 50000
N_EDGES = 800000
D_FEAT = 128

BLK = 1024
NP = 50176
NBLK = NP // BLK
HHALF = NP // 2
TRASH = 128
ACC_ROWS = HHALF + TRASH
ZSTRIPE = ACC_ROWS // 16
OSTRIPE = HHALF // 16
EPT = 50176
NE_PAD = 16 * EPT
EPAD = NE_PAD - N_EDGES
CHUNK = 112
NCH = EPT // CHUNK

BN = 1.0 / math.sqrt(1.0 + 1e-5)

f32 = jnp.float32


SLOTS = 4


def _sc_dloc_body(dst_hbm, dloc0_hbm, dloc1_hbm, buf):
    cid = lax.axis_index("c")
    sid = lax.axis_index("s")
    base = cid * HHALF
    span = EPT
    pltpu.sync_copy(dst_hbm.at[pl.ds(sid * span, span)], buf)

    @pl.loop(0, span // 16)
    def _(i):
        v = buf[pl.ds(i * 16, 16)]
        loc = v - base
        ok = (loc >= 0) & (loc < HHALF)
        buf[pl.ds(i * 16, 16)] = jnp.where(ok, loc, HHALF + (v & (TRASH - 1)))

    @pl.when(cid == 0)
    def _():
        pltpu.sync_copy(buf, dloc0_hbm.at[pl.ds(sid * span, span)])

    @pl.when(cid == 1)
    def _():
        pltpu.sync_copy(buf, dloc1_hbm.at[pl.ds(sid * span, span)])


@functools.cache
def _sc_dloc_kernel():
    return pl.kernel(
        _sc_dloc_body,
        out_type=[jax.ShapeDtypeStruct((NE_PAD,), jnp.int32)] * 2,
        mesh=plsc.VectorSubcoreMesh(core_axis_name="c", subcore_axis_name="s"),
        scratch_types=[pltpu.VMEM((EPT,), jnp.int32)],
        compiler_params=pltpu.CompilerParams(use_tc_tiling_on_sc=False),
    )


def _sc_agg_body(r_hbm, src_hbm, dloc0_hbm, dloc1_hbm, zeros_hbm, out_hbm,
                 sidx, didx, rows, acc, lsem, dsem, gsem, ssem):
    cid = lax.axis_index("c")
    sid = lax.axis_index("s")
    pltpu.sync_copy(zeros_hbm, acc.at[pl.ds(sid * ZSTRIPE, ZSTRIPE)])
    start = sid * EPT
    plsc.subcore_barrier()

    bsl = [pl.ds(b * CHUNK, CHUNK) for b in range(SLOTS)]

    def _issue_idx(j, b):
        esl = pl.ds(start + (j * SLOTS + b) * CHUNK, CHUNK)
        ld = pltpu.async_copy(src_hbm.at[esl], sidx.at[bsl[b]], lsem.at[b])

        @pl.when(cid == 0)
        def _():
            pltpu.async_copy(dloc0_hbm.at[esl], didx.at[bsl[b]], dsem.at[b])

        @pl.when(cid == 1)
        def _():
            pltpu.async_copy(dloc1_hbm.at[esl], didx.at[bsl[b]], dsem.at[b])

        dd = pltpu.make_async_copy(dloc0_hbm.at[esl], didx.at[bsl[b]],
                                   dsem.at[b])
        return ld, dd

    def _gwait_scat(b):
        pltpu.make_async_copy(r_hbm.at[sidx.at[bsl[b]]], rows.at[bsl[b]],
                              gsem.at[b]).wait()
        pltpu.make_async_copy(dloc0_hbm.at[pl.ds(0, CHUNK)], didx.at[bsl[b]],
                              dsem.at[b]).wait()
        pltpu.async_copy(rows.at[bsl[b]], acc.at[didx.at[bsl[b]]],
                         ssem.at[b], add=True)

    def _swait(b):
        pltpu.make_async_copy(rows.at[bsl[b]], acc.at[didx.at[bsl[b]]],
                              ssem.at[b]).wait()

    prime = [_issue_idx(0, b) for b in range(SLOTS)]
    for b in range(SLOTS):
        prime[b][0].wait()
        pltpu.async_copy(r_hbm.at[sidx.at[bsl[b]]], rows.at[bsl[b]],
                         gsem.at[b])

    @pl.loop(1, NCH // SLOTS)
    def _chunk_iter(j):
        for b in range(SLOTS):
            _gwait_scat(b)
        lds = []
        for b in range(SLOTS):
            _swait(b)
            lds.append(_issue_idx(j, b)[0])
        for b in range(SLOTS):
            lds[b].wait()
            pltpu.async_copy(r_hbm.at[sidx.at[bsl[b]]], rows.at[bsl[b]],
                             gsem.at[b])

    for b in range(SLOTS):
        _gwait_scat(b)
    for b in range(SLOTS):
        _swait(b)

    plsc.subcore_barrier()
    pltpu.sync_copy(acc.at[pl.ds(sid * OSTRIPE, OSTRIPE)],
                    out_hbm.at[pl.ds(cid * HHALF + sid * OSTRIPE, OSTRIPE)])


@functools.cache
def _sc_agg_kernel():
    return pl.kernel(
        _sc_agg_body,
        out_type=jax.ShapeDtypeStruct((NP, EMB), f32),
        mesh=plsc.VectorSubcoreMesh(core_axis_name="c", subcore_axis_name="s"),
        scratch_types=[
            pltpu.VMEM((SLOTS * CHUNK,), jnp.int32),
            pltpu.VMEM((SLOTS * CHUNK,), jnp.int32),
            pltpu.VMEM((SLOTS * CHUNK, EMB), f32),
            pltpu.VMEM_SHARED((ACC_ROWS, EMB), f32),
            pltpu.SemaphoreType.DMA((SLOTS,)),
            pltpu.SemaphoreType.DMA((SLOTS,)),
            pltpu.SemaphoreType.DMA((SLOTS,)),
            pltpu.SemaphoreType.DMA((SLOTS,)),
        ],
        compiler_params=pltpu.CompilerParams(use_tc_tiling_on_sc=False),
    )


def _sc_agg(r, src2d, dloc0, dloc1, zeros):
    return _sc_agg_kernel()(r, src2d, dloc0, dloc1, zeros)


def _onehot(batch_blk):
    gi = lax.broadcasted_iota(jnp.int32, (BLK, NG), 1)
    return (batch_blk[:, None] == gi).astype(f32)


def _enc_body(x_ref, b3_ref, W_ref, b_ref, hin_ref, r_ref, vnsum_ref):
    i = pl.program_id(0)
    hin = jnp.dot(x_ref[...], W_ref[...], preferred_element_type=f32) + b_ref[...]
    hin_ref[...] = hin
    r_ref[...] = jnp.maximum(hin, 0.0)
    oh = _onehot(b3_ref[0, 0])

    @pl.when(i == 0)
    def _():
        vnsum_ref[...] = jnp.zeros_like(vnsum_ref)

    vnsum_ref[...] += lax.dot_general(oh, hin, (((0,), (0,)), ((), ())),
                                      preferred_element_type=f32)


def _a_body(h_ref, b3_ref, vn_ref, hin_ref, r_ref, vnsum_ref):
    i = pl.program_id(0)
    oh = _onehot(b3_ref[0, 0])
    hin = h_ref[...] + jnp.dot(oh, vn_ref[...], preferred_element_type=f32)
    hin_ref[...] = hin
    r_ref[...] = jnp.maximum(hin, 0.0)

    @pl.when(i == 0)
    def _():
        vnsum_ref[...] = jnp.zeros_like(vnsum_ref)

    vnsum_ref[...] += lax.dot_general(oh, hin, (((0,), (0,)), ((), ())),
                                      preferred_element_type=f32)


def _mlp(z, W1, b1, W2, b2):
    z1 = jnp.dot(z, W1, preferred_element_type=f32) + b1
    z2 = jnp.maximum(z1 * BN, 0.0)
    z3 = jnp.dot(z2, W2, preferred_element_type=f32) + b2
    return z3 * BN


def _b_body(hin_ref, agg_ref, W1_ref, b1_ref, W2_ref, b2_ref, h_ref):
    h = _mlp(hin_ref[...] + agg_ref[...],
             W1_ref[...], b1_ref[...], W2_ref[...], b2_ref[...])
    h_ref[...] = jnp.maximum(h, 0.0)


def _b_last_body(hin_ref, agg_ref, b3_ref, W1_ref, b1_ref, W2_ref, b2_ref,
                 pool_ref):
    i = pl.program_id(0)
    h = _mlp(hin_ref[...] + agg_ref[...],
             W1_ref[...], b1_ref[...], W2_ref[...], b2_ref[...])
    oh = _onehot(b3_ref[0, 0])

    @pl.when(i == 0)
    def _():
        pool_ref[...] = jnp.zeros_like(pool_ref)

    pool_ref[...] += lax.dot_general(oh, h, (((0,), (0,)), ((), ())),
                                     preferred_element_type=f32)


def _c_body(vnsum_ref, vn_ref, W1_ref, b1_ref, W2_ref, b2_ref, out_ref):
    v = _mlp(vnsum_ref[...] + vn_ref[...],
             W1_ref[...], b1_ref[...], W2_ref[...], b2_ref[...])
    out_ref[...] = jnp.maximum(v, 0.0)


def _head_body(pool_ref, Wp_ref, bp_ref, llp_ref, Wl_ref, bl_ref,
               io_ref, Wio_ref, bio_ref, Wc1_ref, Wc2_ref, Wc3_ref, bc_ref,
               out_ref):
    hg = jnp.dot(pool_ref[...], Wp_ref[...], preferred_element_type=f32) \
        + bp_ref[...]
    lvl = jnp.maximum((llp_ref[...] * Wl_ref[...] + bl_ref[...]) * BN, 0.0)
    io = jnp.maximum((io_ref[...] * Wio_ref[...] + bio_ref[...]) * BN, 0.0)
    comb = (jnp.dot(hg, Wc1_ref[...], preferred_element_type=f32)
            + jnp.dot(io, Wc2_ref[...], preferred_element_type=f32)
            + jnp.dot(lvl, Wc3_ref[...], preferred_element_type=f32)
            + bc_ref[...])
    out_ref[...] = jnp.maximum(comb * BN, 0.0)


def _nspec(cols):
    return pl.BlockSpec((BLK, cols), lambda i: (i, 0))


def _wspec(r, c):
    return pl.BlockSpec((r, c), lambda i: (0, 0))


_B3SPEC = pl.BlockSpec((1, 1, BLK), lambda i: (i, 0, 0))
_GSPEC = pl.BlockSpec((NG, EMB), lambda i: (0, 0))

_node2 = [jax.ShapeDtypeStruct((NP, EMB), f32)] * 2
_gout = jax.ShapeDtypeStruct((NG, EMB), f32)

_enc_call = pl.pallas_call(
    _enc_body, grid=(NBLK,),
    in_specs=[_nspec(D_FEAT), _B3SPEC, _wspec(D_FEAT, EMB), _wspec(1, EMB)],
    out_specs=[_nspec(EMB), _nspec(EMB), _GSPEC],
    out_shape=_node2 + [_gout],
)

_a_call = pl.pallas_call(
    _a_body, grid=(NBLK,),
    in_specs=[_nspec(EMB), _B3SPEC, _GSPEC],
    out_specs=[_nspec(EMB), _nspec(EMB), _GSPEC],
    out_shape=_node2 + [_gout],
)

_b_call = pl.pallas_call(
    _b_body, grid=(NBLK,),
    in_specs=[_nspec(EMB), _nspec(EMB), _wspec(EMB, 2 * EMB),
              _wspec(1, 2 * EMB), _wspec(2 * EMB, EMB), _wspec(1, EMB)],
    out_specs=[_nspec(EMB)],
    out_shape=[jax.ShapeDtypeStruct((NP, EMB), f32)],
)

_b_last_call = pl.pallas_call(
    _b_last_body, grid=(NBLK,),
    in_specs=[_nspec(EMB), _nspec(EMB), _B3SPEC, _wspec(EMB, 2 * EMB),
              _wspec(1, 2 * EMB), _wspec(2 * EMB, EMB), _wspec(1, EMB)],
    out_specs=[_GSPEC],
    out_shape=[_gout],
)

_c_call = pl.pallas_call(
    _c_body,
    out_shape=[_gout],
)

_head_call = pl.pallas_call(
    _head_body,
    out_shape=jax.ShapeDtypeStruct((NG, EMB), f32),
)


def kernel(x, edge_index, batch, len_longest_path, io_ratio, W_enc, b_enc,
           gin_W1, gin_b1, gin_W2, gin_b2, vn_W1, vn_b1, vn_W2, vn_b2,
           W_pred, b_pred, W_level, b_level, W_io, b_io, W_comb, b_comb):
    x_p = jnp.pad(x, ((0, NP - N_NODES), (0, 0)))
    batch_p = jnp.pad(batch.astype(jnp.int32), (0, NP - N_NODES),
                      constant_values=NG).reshape(NBLK, 1, BLK)
    src_flat = jnp.pad(edge_index[0].astype(jnp.int32), (0, EPAD))
    dst_flat = jnp.pad(edge_index[1].astype(jnp.int32), (0, EPAD),
                       constant_values=NP - 1)
    zeros = jnp.zeros((ZSTRIPE, EMB), f32)
    dloc0, dloc1 = _sc_dloc_kernel()(dst_flat)

    h_in, r, vnsum = _enc_call(x_p, batch_p, W_enc, b_enc.reshape(1, EMB))
    vn = jnp.zeros((NG, EMB), f32)
    for l in range(NL):
        agg = _sc_agg(r, src_flat, dloc0, dloc1, zeros)
        if l < NL - 1:
            (vn,) = _c_call(vnsum, vn, vn_W1[l], vn_b1[l].reshape(1, 2 * EMB),
                            vn_W2[l], vn_b2[l].reshape(1, EMB))
            (h,) = _b_call(h_in, agg, gin_W1[l], gin_b1[l].reshape(1, 2 * EMB),
                           gin_W2[l], gin_b2[l].reshape(1, EMB))
            h_in, r, vnsum = _a_call(h, batch_p, vn)
        else:
            (pool,) = _b_last_call(h_in, agg, batch_p, gin_W1[l],
                                   gin_b1[l].reshape(1, 2 * EMB), gin_W2[l],
                                   gin_b2[l].reshape(1, EMB))
    return _head_call(pool, W_pred, b_pred.reshape(1, EMB),
                      len_longest_path.reshape(NG, 1), W_level,
                      b_level.reshape(1, EMB), io_ratio.reshape(NG, 1),
                      W_io, b_io.reshape(1, EMB), W_comb[:EMB],
                      W_comb[EMB:2 * EMB], W_comb[2 * EMB:],
                      b_comb.reshape(1, EMB))


# speedup vs baseline: 8.6022x; 1.4150x over previous
import functools
import math

import jax
import jax.numpy as jnp
from jax import lax
from jax.experimental import pallas as pl
from jax.experimental.pallas import tpu as pltpu
from jax.experimental.pallas import tpu_sc as plsc

EMB = 64
NL = 5
NG = 512
N_NODES = 50000
N_EDGES = 800000
D_FEAT = 128

BLK = 1024
NP = 50176
NBLK = NP // BLK
HHALF = NP // 2
TRASH = 128
ACC_ROWS = HHALF + TRASH
ZSTRIPE = ACC_ROWS // 16
OSTRIPE = HHALF // 16
EPT = 50176
NE_PAD = 16 * EPT
EPAD = NE_PAD - N_EDGES
CHUNK = 112
NCH = EPT // CHUNK

BN = 1.0 / math.sqrt(1.0 + 1e-5)

f32 = jnp.float32


SLOTS = 4


IGN = -1


def _sc_dloc_body(src_hbm, dst_hbm, sloc0_hbm, dloc0_hbm, sloc1_hbm,
                  dloc1_hbm, sbuf, dbuf):
    cid = lax.axis_index("c")
    sid = lax.axis_index("s")
    base = cid * HHALF
    span = EPT
    pltpu.sync_copy(src_hbm.at[pl.ds(sid * span, span)], sbuf)
    pltpu.sync_copy(dst_hbm.at[pl.ds(sid * span, span)], dbuf)

    @pl.loop(0, span // 16)
    def _(i):
        v = dbuf[pl.ds(i * 16, 16)]
        s = sbuf[pl.ds(i * 16, 16)]
        loc = v - base
        ok = (loc >= 0) & (loc < HHALF)
        dbuf[pl.ds(i * 16, 16)] = jnp.where(ok, loc, IGN)
        sbuf[pl.ds(i * 16, 16)] = jnp.where(ok, s, IGN)

    @pl.when(cid == 0)
    def _():
        pltpu.sync_copy(sbuf, sloc0_hbm.at[pl.ds(sid * span, span)])
        pltpu.sync_copy(dbuf, dloc0_hbm.at[pl.ds(sid * span, span)])

    @pl.when(cid == 1)
    def _():
        pltpu.sync_copy(sbuf, sloc1_hbm.at[pl.ds(sid * span, span)])
        pltpu.sync_copy(dbuf, dloc1_hbm.at[pl.ds(sid * span, span)])


@functools.cache
def _sc_dloc_kernel():
    return pl.kernel(
        _sc_dloc_body,
        out_type=[jax.ShapeDtypeStruct((NE_PAD,), jnp.int32)] * 4,
        mesh=plsc.VectorSubcoreMesh(core_axis_name="c", subcore_axis_name="s"),
        scratch_types=[pltpu.VMEM((EPT,), jnp.int32),
                       pltpu.VMEM((EPT,), jnp.int32)],
        compiler_params=pltpu.CompilerParams(use_tc_tiling_on_sc=False),
    )


def _sc_agg_body(r_hbm, sloc0_hbm, sloc1_hbm, dloc0_hbm, dloc1_hbm,
                 zeros_hbm, out_hbm,
                 sidx, didx, rows, acc, lsem, dsem, gsem, ssem):
    cid = lax.axis_index("c")
    sid = lax.axis_index("s")
    pltpu.sync_copy(zeros_hbm, acc.at[pl.ds(sid * ZSTRIPE, ZSTRIPE)])
    start = sid * EPT
    plsc.subcore_barrier()

    bsl = [pl.ds(b * CHUNK, CHUNK) for b in range(SLOTS)]

    def _gsl(b):
        return plsc.Indices(sidx.at[bsl[b]], ignored_value=IGN)

    def _ssl(b):
        return plsc.Indices(didx.at[bsl[b]], ignored_value=IGN)

    def _issue_idx(j, b):
        esl = pl.ds(start + (j * SLOTS + b) * CHUNK, CHUNK)

        @pl.when(cid == 0)
        def _():
            pltpu.async_copy(sloc0_hbm.at[esl], sidx.at[bsl[b]], lsem.at[b])
            pltpu.async_copy(dloc0_hbm.at[esl], didx.at[bsl[b]], dsem.at[b])

        @pl.when(cid == 1)
        def _():
            pltpu.async_copy(sloc1_hbm.at[esl], sidx.at[bsl[b]], lsem.at[b])
            pltpu.async_copy(dloc1_hbm.at[esl], didx.at[bsl[b]], dsem.at[b])

        return pltpu.make_async_copy(sloc0_hbm.at[esl], sidx.at[bsl[b]],
                                     lsem.at[b])

    def _gwait_scat(b):
        pltpu.make_async_copy(r_hbm.at[_gsl(b)], rows.at[bsl[b]],
                              gsem.at[b]).wait()
        pltpu.make_async_copy(dloc0_hbm.at[pl.ds(0, CHUNK)], didx.at[bsl[b]],
                              dsem.at[b]).wait()
        pltpu.async_copy(rows.at[bsl[b]], acc.at[_ssl(b)],
                         ssem.at[b], add=True)

    def _swait(b):
        pltpu.make_async_copy(rows.at[bsl[b]], acc.at[_ssl(b)],
                              ssem.at[b]).wait()

    prime = [_issue_idx(0, b) for b in range(SLOTS)]
    for b in range(SLOTS):
        prime[b].wait()
        pltpu.async_copy(r_hbm.at[_gsl(b)], rows.at[bsl[b]], gsem.at[b])

    @pl.loop(1, NCH // SLOTS)
    def _chunk_iter(j):
        for b in range(SLOTS):
            _gwait_scat(b)
        lds = []
        for b in range(SLOTS):
            _swait(b)
            lds.append(_issue_idx(j, b))
        for b in range(SLOTS):
            lds[b].wait()
            pltpu.async_copy(r_hbm.at[_gsl(b)], rows.at[bsl[b]], gsem.at[b])

    for b in range(SLOTS):
        _gwait_scat(b)
    for b in range(SLOTS):
        _swait(b)

    plsc.subcore_barrier()
    pltpu.sync_copy(acc.at[pl.ds(sid * OSTRIPE, OSTRIPE)],
                    out_hbm.at[pl.ds(cid * HHALF + sid * OSTRIPE, OSTRIPE)])


@functools.cache
def _sc_agg_kernel():
    return pl.kernel(
        _sc_agg_body,
        out_type=jax.ShapeDtypeStruct((NP, EMB), f32),
        mesh=plsc.VectorSubcoreMesh(core_axis_name="c", subcore_axis_name="s"),
        scratch_types=[
            pltpu.VMEM((SLOTS * CHUNK,), jnp.int32),
            pltpu.VMEM((SLOTS * CHUNK,), jnp.int32),
            pltpu.VMEM((SLOTS * CHUNK, EMB), f32),
            pltpu.VMEM_SHARED((ACC_ROWS, EMB), f32),
            pltpu.SemaphoreType.DMA((SLOTS,)),
            pltpu.SemaphoreType.DMA((SLOTS,)),
            pltpu.SemaphoreType.DMA((SLOTS,)),
            pltpu.SemaphoreType.DMA((SLOTS,)),
        ],
        compiler_params=pltpu.CompilerParams(use_tc_tiling_on_sc=False),
    )


def _sc_agg(r, sloc0, sloc1, dloc0, dloc1, zeros):
    return _sc_agg_kernel()(r, sloc0, sloc1, dloc0, dloc1, zeros)


def _onehot(batch_blk):
    gi = lax.broadcasted_iota(jnp.int32, (BLK, NG), 1)
    return (batch_blk[:, None] == gi).astype(f32)


def _enc_body(x_ref, b3_ref, W_ref, b_ref, hin_ref, r_ref, vnsum_ref):
    i = pl.program_id(0)
    hin = jnp.dot(x_ref[...], W_ref[...], preferred_element_type=f32) + b_ref[...]
    hin_ref[...] = hin
    r_ref[...] = jnp.maximum(hin, 0.0)
    oh = _onehot(b3_ref[0, 0])

    @pl.when(i == 0)
    def _():
        vnsum_ref[...] = jnp.zeros_like(vnsum_ref)

    vnsum_ref[...] += lax.dot_general(oh, hin, (((0,), (0,)), ((), ())),
                                      preferred_element_type=f32)


def _a_body(h_ref, b3_ref, vn_ref, hin_ref, r_ref, vnsum_ref):
    i = pl.program_id(0)
    oh = _onehot(b3_ref[0, 0])
    hin = h_ref[...] + jnp.dot(oh, vn_ref[...], preferred_element_type=f32)
    hin_ref[...] = hin
    r_ref[...] = jnp.maximum(hin, 0.0)

    @pl.when(i == 0)
    def _():
        vnsum_ref[...] = jnp.zeros_like(vnsum_ref)

    vnsum_ref[...] += lax.dot_general(oh, hin, (((0,), (0,)), ((), ())),
                                      preferred_element_type=f32)


def _mlp(z, W1, b1, W2, b2):
    z1 = jnp.dot(z, W1, preferred_element_type=f32) + b1
    z2 = jnp.maximum(z1 * BN, 0.0)
    z3 = jnp.dot(z2, W2, preferred_element_type=f32) + b2
    return z3 * BN


def _b_body(hin_ref, agg_ref, W1_ref, b1_ref, W2_ref, b2_ref, h_ref):
    h = _mlp(hin_ref[...] + agg_ref[...],
             W1_ref[...], b1_ref[...], W2_ref[...], b2_ref[...])
    h_ref[...] = jnp.maximum(h, 0.0)


def _b_last_body(hin_ref, agg_ref, b3_ref, W1_ref, b1_ref, W2_ref, b2_ref,
                 pool_ref):
    i = pl.program_id(0)
    h = _mlp(hin_ref[...] + agg_ref[...],
             W1_ref[...], b1_ref[...], W2_ref[...], b2_ref[...])
    oh = _onehot(b3_ref[0, 0])

    @pl.when(i == 0)
    def _():
        pool_ref[...] = jnp.zeros_like(pool_ref)

    pool_ref[...] += lax.dot_general(oh, h, (((0,), (0,)), ((), ())),
                                     preferred_element_type=f32)


def _c_body(vnsum_ref, vn_ref, W1_ref, b1_ref, W2_ref, b2_ref, out_ref):
    v = _mlp(vnsum_ref[...] + vn_ref[...],
             W1_ref[...], b1_ref[...], W2_ref[...], b2_ref[...])
    out_ref[...] = jnp.maximum(v, 0.0)


def _head_body(pool_ref, Wp_ref, bp_ref, llp_ref, Wl_ref, bl_ref,
               io_ref, Wio_ref, bio_ref, Wc1_ref, Wc2_ref, Wc3_ref, bc_ref,
               out_ref):
    hg = jnp.dot(pool_ref[...], Wp_ref[...], preferred_element_type=f32) \
        + bp_ref[...]
    lvl = jnp.maximum((llp_ref[...] * Wl_ref[...] + bl_ref[...]) * BN, 0.0)
    io = jnp.maximum((io_ref[...] * Wio_ref[...] + bio_ref[...]) * BN, 0.0)
    comb = (jnp.dot(hg, Wc1_ref[...], preferred_element_type=f32)
            + jnp.dot(io, Wc2_ref[...], preferred_element_type=f32)
            + jnp.dot(lvl, Wc3_ref[...], preferred_element_type=f32)
            + bc_ref[...])
    out_ref[...] = jnp.maximum(comb * BN, 0.0)


def _nspec(cols):
    return pl.BlockSpec((BLK, cols), lambda i: (i, 0))


def _wspec(r, c):
    return pl.BlockSpec((r, c), lambda i: (0, 0))


_B3SPEC = pl.BlockSpec((1, 1, BLK), lambda i: (i, 0, 0))
_GSPEC = pl.BlockSpec((NG, EMB), lambda i: (0, 0))

_node2 = [jax.ShapeDtypeStruct((NP, EMB), f32)] * 2
_gout = jax.ShapeDtypeStruct((NG, EMB), f32)

_enc_call = pl.pallas_call(
    _enc_body, grid=(NBLK,),
    in_specs=[_nspec(D_FEAT), _B3SPEC, _wspec(D_FEAT, EMB), _wspec(1, EMB)],
    out_specs=[_nspec(EMB), _nspec(EMB), _GSPEC],
    out_shape=_node2 + [_gout],
)

_a_call = pl.pallas_call(
    _a_body, grid=(NBLK,),
    in_specs=[_nspec(EMB), _B3SPEC, _GSPEC],
    out_specs=[_nspec(EMB), _nspec(EMB), _GSPEC],
    out_shape=_node2 + [_gout],
)

_b_call = pl.pallas_call(
    _b_body, grid=(NBLK,),
    in_specs=[_nspec(EMB), _nspec(EMB), _wspec(EMB, 2 * EMB),
              _wspec(1, 2 * EMB), _wspec(2 * EMB, EMB), _wspec(1, EMB)],
    out_specs=[_nspec(EMB)],
    out_shape=[jax.ShapeDtypeStruct((NP, EMB), f32)],
)

_b_last_call = pl.pallas_call(
    _b_last_body, grid=(NBLK,),
    in_specs=[_nspec(EMB), _nspec(EMB), _B3SPEC, _wspec(EMB, 2 * EMB),
              _wspec(1, 2 * EMB), _wspec(2 * EMB, EMB), _wspec(1, EMB)],
    out_specs=[_GSPEC],
    out_shape=[_gout],
)

_c_call = pl.pallas_call(
    _c_body,
    out_shape=[_gout],
)

_head_call = pl.pallas_call(
    _head_body,
    out_shape=jax.ShapeDtypeStruct((NG, EMB), f32),
)


def kernel(x, edge_index, batch, len_longest_path, io_ratio, W_enc, b_enc,
           gin_W1, gin_b1, gin_W2, gin_b2, vn_W1, vn_b1, vn_W2, vn_b2,
           W_pred, b_pred, W_level, b_level, W_io, b_io, W_comb, b_comb):
    x_p = jnp.pad(x, ((0, NP - N_NODES), (0, 0)))
    batch_p = jnp.pad(batch.astype(jnp.int32), (0, NP - N_NODES),
                      constant_values=NG).reshape(NBLK, 1, BLK)
    src_flat = jnp.pad(edge_index[0].astype(jnp.int32), (0, EPAD))
    dst_flat = jnp.pad(edge_index[1].astype(jnp.int32), (0, EPAD),
                       constant_values=IGN)
    zeros = jnp.zeros((ZSTRIPE, EMB), f32)
    sloc0, dloc0, sloc1, dloc1 = _sc_dloc_kernel()(src_flat, dst_flat)

    h_in, r, vnsum = _enc_call(x_p, batch_p, W_enc, b_enc.reshape(1, EMB))
    vn = jnp.zeros((NG, EMB), f32)
    for l in range(NL):
        agg = _sc_agg(r, sloc0, sloc1, dloc0, dloc1, zeros)
        if l < NL - 1:
            (vn,) = _c_call(vnsum, vn, vn_W1[l], vn_b1[l].reshape(1, 2 * EMB),
                            vn_W2[l], vn_b2[l].reshape(1, EMB))
            (h,) = _b_call(h_in, agg, gin_W1[l], gin_b1[l].reshape(1, 2 * EMB),
                           gin_W2[l], gin_b2[l].reshape(1, EMB))
            h_in, r, vnsum = _a_call(h, batch_p, vn)
        else:
            (pool,) = _b_last_call(h_in, agg, batch_p, gin_W1[l],
                                   gin_b1[l].reshape(1, 2 * EMB), gin_W2[l],
                                   gin_b2[l].reshape(1, EMB))
    return _head_call(pool, W_pred, b_pred.reshape(1, EMB),
                      len_longest_path.reshape(NG, 1), W_level,
                      b_level.reshape(1, EMB), io_ratio.reshape(NG, 1),
                      W_io, b_io.reshape(1, EMB), W_comb[:EMB],
                      W_comb[EMB:2 * EMB], W_comb[2 * EMB:],
                      b_comb.reshape(1, EMB))
